# Initial kernel scaffold; baseline (speedup 1.0000x reference)
import jax
import jax.numpy as jnp
from jax.experimental import pallas as pl


def kernel(batch_input, emb, W, b):
    raise NotImplementedError("write your pallas kernel here")


# speedup vs baseline: 2.7802x; 2.7802x over previous
import functools

import jax
import jax.numpy as jnp
from jax import lax
from jax.experimental import pallas as pl
from jax.experimental.pallas import tpu as pltpu
from jax.experimental.pallas import tpu_sc as plsc

B = 16384
S = 50
D = 64
NL = 5

NC = 2
NS = 16
L = 16
NW = NC * NS
BPW = B // NW
CH = 8
ROWS = CH * S
NCHUNK = BPW // CH
NBUF = 2

_VREGS = D // L


def _pool_body(idx_hbm, emb_hbm, out_hbm, idx_v, out_v, buf0, buf1, sem0,
               sem1):
    wid = lax.axis_index("s") * NC + lax.axis_index("c")
    pltpu.sync_copy(idx_hbm.at[wid], idx_v)

    bufs = (buf0, buf1)
    sems = (sem0, sem1)

    def _idx_slice(g):
        return idx_v.at[pl.ds(g * ROWS, ROWS)]

    for k in range(NBUF):
        pltpu.async_copy(emb_hbm.at[_idx_slice(k)], bufs[k], sems[k])

    @pl.loop(0, NCHUNK, step=NBUF)
    def _chunks(g0):
        for k in range(NBUF):
            g = g0 + k
            buf = bufs[k]
            pltpu.make_async_copy(emb_hbm.at[_idx_slice(g)], buf,
                                  sems[k]).wait()

            def _elem(e, _):
                row0 = e * S

                def _row(r, acc):
                    base = row0 + r
                    return tuple(
                        acc[v] + buf[base, pl.ds(v * L, L)]
                        for v in range(_VREGS))

                zero = jnp.zeros((L,), jnp.float32)
                acc = lax.fori_loop(0, S, _row, (zero,) * _VREGS, unroll=10)
                orow = g * CH + e
                for v in range(_VREGS):
                    out_v[orow, pl.ds(v * L, L)] = acc[v]
                return 0

            lax.fori_loop(0, CH, _elem, 0)

            @pl.when(g + NBUF < NCHUNK)
            def _():
                pltpu.async_copy(emb_hbm.at[_idx_slice(g + NBUF)], buf,
                                 sems[k])

    pltpu.sync_copy(out_v, out_hbm.at[pl.ds(wid * BPW, BPW)])


@functools.cache
def _make_pool():
    return pl.kernel(
        _pool_body,
        out_type=jax.ShapeDtypeStruct((B, D), jnp.float32),
        mesh=plsc.VectorSubcoreMesh(core_axis_name="c", subcore_axis_name="s",
                                    num_cores=NC, num_subcores=NS),
        compiler_params=pltpu.CompilerParams(use_tc_tiling_on_sc=False),
        scratch_types=[
            pltpu.VMEM((NCHUNK * ROWS,), jnp.int32),
            pltpu.VMEM((BPW, D), jnp.float32),
            pltpu.VMEM((ROWS, D), jnp.float32),
            pltpu.VMEM((ROWS, D), jnp.float32),
            pltpu.SemaphoreType.DMA,
            pltpu.SemaphoreType.DMA,
        ],
    )


_BT = 2048


def _cls_body(x_ref, w_ref, b_ref, o_ref):
    x = x_ref[...]
    w = w_ref[...]
    logits = jnp.dot(x, w, preferred_element_type=jnp.float32)
    logits = logits * (1.0 / S) + b_ref[...]
    m = jnp.max(logits, axis=1, keepdims=True)
    ex = jnp.exp(logits - m)
    lse = jnp.log(jnp.sum(ex, axis=1, keepdims=True)) + m
    o_ref[...] = logits - lse


_cls = pl.pallas_call(
    _cls_body,
    grid=(B // _BT,),
    in_specs=[
        pl.BlockSpec((_BT, D), lambda i: (i, 0)),
        pl.BlockSpec((D, NL), lambda i: (0, 0)),
        pl.BlockSpec((1, NL), lambda i: (0, 0)),
    ],
    out_specs=pl.BlockSpec((_BT, NL), lambda i: (i, 0)),
    out_shape=jax.ShapeDtypeStruct((B, NL), jnp.float32),
)


def kernel(batch_input, emb, W, b):
    idx = batch_input.reshape(NW, NCHUNK * ROWS)
    pooled = _make_pool()(idx, emb)
    return _cls(pooled, W, b.reshape(1, NL))
